# baseline (speedup 1.0000x reference)
import jax
import jax.numpy as jnp
from jax.experimental import pallas as pl
from jax.experimental.pallas import tpu as pltpu

_SCALE = 80.0
_R2 = 40.0 * 40.0
_BIG = 1e10
_TN = 2048


def _chamfer_kernel(p_ref, g_ref, out_ref):
    N = p_ref.shape[1]
    M = g_ref.shape[1]

    gt = jnp.transpose(g_ref[0], (1, 0))
    gx = gt[0:1, :] * _SCALE
    gy = gt[1:2, :] * _SCALE
    gz = gt[2:3, :] * _SCALE
    bb = gx * gx + gy * gy + gz * gz
    mg = bb < _R2
    bbm = jnp.where(mg, bb, bb + _BIG)

    def neg2bf16(v):
        return (-2.0 * v.astype(jnp.bfloat16).astype(jnp.float32)
                ).astype(jnp.bfloat16)

    g2 = jnp.concatenate(
        [neg2bf16(gx), neg2bf16(gy), neg2bf16(gz)], axis=0)

    def body(j, carry):
        cmin_u, cmin_m, rsum_u, rsum_m, cnt_p = carry
        p_blk = p_ref[0, pl.ds(j * _TN, _TN), :] * _SCALE
        px = p_blk[:, 0:1]
        py = p_blk[:, 1:2]
        pz = p_blk[:, 2:3]
        aa = px * px + py * py + pz * pz
        mp = aa < _R2
        aam = jnp.where(mp, aa, aa + _BIG)

        p_r = p_blk.astype(jnp.bfloat16)
        ab2 = jax.lax.dot_general(
            p_r, g2, (((1,), (0,)), ((), ())),
            preferred_element_type=jnp.float32)

        rmin_u = jnp.maximum(
            aa + jnp.min(bb + ab2, axis=1, keepdims=True), 0.0)
        rmin_m = jnp.maximum(
            aa + jnp.min(bbm + ab2, axis=1, keepdims=True), 0.0)
        cmin_u = jnp.minimum(cmin_u, jnp.min(aa + ab2, axis=0, keepdims=True))
        cmin_m = jnp.minimum(cmin_m, jnp.min(aam + ab2, axis=0, keepdims=True))

        rsum_u = rsum_u + jnp.sum(rmin_u)
        rsum_m = rsum_m + jnp.sum(jnp.where(mp, rmin_m, 0.0))
        cnt_p = cnt_p + jnp.sum(mp.astype(jnp.float32))
        return cmin_u, cmin_m, rsum_u, rsum_m, cnt_p

    init = (
        jnp.full((1, M), _BIG, jnp.float32),
        jnp.full((1, M), _BIG, jnp.float32),
        jnp.float32(0.0),
        jnp.float32(0.0),
        jnp.float32(0.0),
    )
    cmin_u, cmin_m, rsum_u, rsum_m, cnt_p = jax.lax.fori_loop(
        0, N // _TN, body, init)

    cmin_u = jnp.maximum(bb + cmin_u, 0.0)
    cmin_m = jnp.maximum(bb + cmin_m, 0.0)
    sum_c_u = jnp.sum(cmin_u)
    sum_c_m = jnp.sum(jnp.where(mg, cmin_m, 0.0))
    cnt_g = jnp.sum(mg.astype(jnp.float32))

    non_filtered = rsum_u / N + sum_c_u / M
    filtered = (rsum_m / jnp.maximum(cnt_p, 1.0)
                + sum_c_m / jnp.maximum(cnt_g, 1.0))
    loss = (0.7 * filtered + 0.3 * non_filtered) / pl.num_programs(0)

    @pl.when(pl.program_id(0) == 0)
    def _first():
        out_ref[:, :, :] = jnp.broadcast_to(loss, (1, 1, 1))

    @pl.when(pl.program_id(0) != 0)
    def _rest():
        out_ref[:, :, :] = out_ref[:, :, :] + loss


def kernel(image_pred, image_gt):
    B, N, _ = image_pred.shape
    M = image_gt.shape[1]

    per_batch = pl.pallas_call(
        _chamfer_kernel,
        grid=(B,),
        in_specs=[
            pl.BlockSpec((1, N, 3), lambda b: (b, 0, 0)),
            pl.BlockSpec((1, M, 3), lambda b: (b, 0, 0)),
        ],
        out_specs=pl.BlockSpec((1, 1, 1), lambda b: (0, 0, 0)),
        out_shape=jax.ShapeDtypeStruct((1, 1, 1), jnp.float32),
        compiler_params=pltpu.CompilerParams(
            dimension_semantics=("arbitrary",)),
    )(image_pred, image_gt)
    return per_batch.reshape(())


# speedup vs baseline: 1.0410x; 1.0410x over previous
import jax
import jax.numpy as jnp
from jax.experimental import pallas as pl
from jax.experimental.pallas import tpu as pltpu

_SCALE = 80.0
_R2 = 40.0 * 40.0
_BIG = 1e10
_TN = 2048


def _chamfer_kernel(p_ref, gt_ref, out_ref):
    N = p_ref.shape[1]
    M = gt_ref.shape[2]

    gx = gt_ref[0, 0:1, :] * _SCALE
    gy = gt_ref[0, 1:2, :] * _SCALE
    gz = gt_ref[0, 2:3, :] * _SCALE
    bb = gx * gx + gy * gy + gz * gz
    mg = bb < _R2
    bbm = jnp.where(mg, bb, bb + _BIG)

    def neg2bf16(v):
        return (-2.0 * v.astype(jnp.bfloat16).astype(jnp.float32)
                ).astype(jnp.bfloat16)

    g2 = jnp.concatenate(
        [neg2bf16(gx), neg2bf16(gy), neg2bf16(gz)], axis=0)

    def body(j, carry):
        cmin_u, cmin_m, rsum_u, rsum_m, cnt_p = carry
        p_blk = p_ref[0, pl.ds(j * _TN, _TN), :] * _SCALE
        px = p_blk[:, 0:1]
        py = p_blk[:, 1:2]
        pz = p_blk[:, 2:3]
        aa = px * px + py * py + pz * pz
        mp = aa < _R2
        aam = jnp.where(mp, aa, aa + _BIG)

        p_r = p_blk.astype(jnp.bfloat16)
        ab2 = jax.lax.dot_general(
            p_r, g2, (((1,), (0,)), ((), ())),
            preferred_element_type=jnp.float32)

        rmin_u = jnp.maximum(
            aa + jnp.min(bb + ab2, axis=1, keepdims=True), 0.0)
        rmin_m = jnp.maximum(
            aa + jnp.min(bbm + ab2, axis=1, keepdims=True), 0.0)
        cmin_u = jnp.minimum(cmin_u, jnp.min(aa + ab2, axis=0, keepdims=True))
        cmin_m = jnp.minimum(cmin_m, jnp.min(aam + ab2, axis=0, keepdims=True))

        rsum_u = rsum_u + jnp.sum(rmin_u)
        rsum_m = rsum_m + jnp.sum(jnp.where(mp, rmin_m, 0.0))
        cnt_p = cnt_p + jnp.sum(mp.astype(jnp.float32))
        return cmin_u, cmin_m, rsum_u, rsum_m, cnt_p

    init = (
        jnp.full((1, M), _BIG, jnp.float32),
        jnp.full((1, M), _BIG, jnp.float32),
        jnp.float32(0.0),
        jnp.float32(0.0),
        jnp.float32(0.0),
    )
    cmin_u, cmin_m, rsum_u, rsum_m, cnt_p = jax.lax.fori_loop(
        0, N // _TN, body, init)

    cmin_u = jnp.maximum(bb + cmin_u, 0.0)
    cmin_m = jnp.maximum(bb + cmin_m, 0.0)
    sum_c_u = jnp.sum(cmin_u)
    sum_c_m = jnp.sum(jnp.where(mg, cmin_m, 0.0))
    cnt_g = jnp.sum(mg.astype(jnp.float32))

    non_filtered = rsum_u / N + sum_c_u / M
    filtered = (rsum_m / jnp.maximum(cnt_p, 1.0)
                + sum_c_m / jnp.maximum(cnt_g, 1.0))
    loss = (0.7 * filtered + 0.3 * non_filtered) / pl.num_programs(0)

    @pl.when(pl.program_id(0) == 0)
    def _first():
        out_ref[:, :, :] = jnp.broadcast_to(loss, (1, 1, 1))

    @pl.when(pl.program_id(0) != 0)
    def _rest():
        out_ref[:, :, :] = out_ref[:, :, :] + loss


def kernel(image_pred, image_gt):
    B, N, _ = image_pred.shape
    M = image_gt.shape[1]
    gt_t = jnp.swapaxes(image_gt, 1, 2)

    per_batch = pl.pallas_call(
        _chamfer_kernel,
        grid=(B,),
        in_specs=[
            pl.BlockSpec((1, N, 3), lambda b: (b, 0, 0)),
            pl.BlockSpec((1, 3, M), lambda b: (b, 0, 0)),
        ],
        out_specs=pl.BlockSpec((1, 1, 1), lambda b: (0, 0, 0)),
        out_shape=jax.ShapeDtypeStruct((1, 1, 1), jnp.float32),
        compiler_params=pltpu.CompilerParams(
            dimension_semantics=("arbitrary",)),
    )(image_pred, gt_t)
    return per_batch.reshape(())


# speedup vs baseline: 1.0771x; 1.0347x over previous
import jax
import jax.numpy as jnp
from jax.experimental import pallas as pl
from jax.experimental.pallas import tpu as pltpu

_SCALE = 80.0
_R2 = 40.0 * 40.0
_BIG = 1e10
_TN = 4096


def _chamfer_kernel(p_ref, gt_ref, out_ref):
    N = p_ref.shape[1]
    M = gt_ref.shape[2]

    gx = gt_ref[0, 0:1, :] * _SCALE
    gy = gt_ref[0, 1:2, :] * _SCALE
    gz = gt_ref[0, 2:3, :] * _SCALE
    bb = gx * gx + gy * gy + gz * gz
    mg = bb < _R2
    bbm = jnp.where(mg, bb, bb + _BIG)

    def neg2bf16(v):
        return (-2.0 * v.astype(jnp.bfloat16).astype(jnp.float32)
                ).astype(jnp.bfloat16)

    g2 = jnp.concatenate(
        [neg2bf16(gx), neg2bf16(gy), neg2bf16(gz)], axis=0)

    def body(j, carry):
        cmin_u, cmin_m, rsum_u, rsum_m, cnt_p = carry
        p_blk = p_ref[0, pl.ds(j * _TN, _TN), :] * _SCALE
        px = p_blk[:, 0:1]
        py = p_blk[:, 1:2]
        pz = p_blk[:, 2:3]
        aa = px * px + py * py + pz * pz
        mp = aa < _R2
        aam = jnp.where(mp, aa, aa + _BIG)

        p_r = p_blk.astype(jnp.bfloat16)
        ab2 = jax.lax.dot_general(
            p_r, g2, (((1,), (0,)), ((), ())),
            preferred_element_type=jnp.float32)

        rmin_u = jnp.maximum(
            aa + jnp.min(bb + ab2, axis=1, keepdims=True), 0.0)
        rmin_m = jnp.maximum(
            aa + jnp.min(bbm + ab2, axis=1, keepdims=True), 0.0)
        cmin_u = jnp.minimum(cmin_u, jnp.min(aa + ab2, axis=0, keepdims=True))
        cmin_m = jnp.minimum(cmin_m, jnp.min(aam + ab2, axis=0, keepdims=True))

        rsum_u = rsum_u + jnp.sum(rmin_u)
        rsum_m = rsum_m + jnp.sum(jnp.where(mp, rmin_m, 0.0))
        cnt_p = cnt_p + jnp.sum(mp.astype(jnp.float32))
        return cmin_u, cmin_m, rsum_u, rsum_m, cnt_p

    init = (
        jnp.full((1, M), _BIG, jnp.float32),
        jnp.full((1, M), _BIG, jnp.float32),
        jnp.float32(0.0),
        jnp.float32(0.0),
        jnp.float32(0.0),
    )
    cmin_u, cmin_m, rsum_u, rsum_m, cnt_p = jax.lax.fori_loop(
        0, N // _TN, body, init)

    cmin_u = jnp.maximum(bb + cmin_u, 0.0)
    cmin_m = jnp.maximum(bb + cmin_m, 0.0)
    sum_c_u = jnp.sum(cmin_u)
    sum_c_m = jnp.sum(jnp.where(mg, cmin_m, 0.0))
    cnt_g = jnp.sum(mg.astype(jnp.float32))

    non_filtered = rsum_u / N + sum_c_u / M
    filtered = (rsum_m / jnp.maximum(cnt_p, 1.0)
                + sum_c_m / jnp.maximum(cnt_g, 1.0))
    loss = (0.7 * filtered + 0.3 * non_filtered) / pl.num_programs(0)

    @pl.when(pl.program_id(0) == 0)
    def _first():
        out_ref[:, :, :] = jnp.broadcast_to(loss, (1, 1, 1))

    @pl.when(pl.program_id(0) != 0)
    def _rest():
        out_ref[:, :, :] = out_ref[:, :, :] + loss


def kernel(image_pred, image_gt):
    B, N, _ = image_pred.shape
    M = image_gt.shape[1]
    gt_t = jnp.swapaxes(image_gt, 1, 2)

    per_batch = pl.pallas_call(
        _chamfer_kernel,
        grid=(B,),
        in_specs=[
            pl.BlockSpec((1, N, 3), lambda b: (b, 0, 0)),
            pl.BlockSpec((1, 3, M), lambda b: (b, 0, 0)),
        ],
        out_specs=pl.BlockSpec((1, 1, 1), lambda b: (0, 0, 0)),
        out_shape=jax.ShapeDtypeStruct((1, 1, 1), jnp.float32),
        compiler_params=pltpu.CompilerParams(
            dimension_semantics=("arbitrary",)),
    )(image_pred, gt_t)
    return per_batch.reshape(())
